# baseline (speedup 1.0000x reference)
import functools

import jax
import jax.numpy as jnp
from jax import lax
from jax.experimental import pallas as pl
from jax.experimental.pallas import tpu as pltpu
from jax.experimental.pallas import tpu_sc as plsc

_N = 10000
_E = 320000
_D = 128
_B = 8
_NPG = _N // _B
_NC = 2
_NS = 16
_NW = _NC * _NS
_EPT = _E // _NW
_K = 80
_NCHUNK = _EPT // _K
_NCHG = _E // _K
_NP = 10240
_RPT = _NP // _NS


def _pre_body(obs_ref, mean_ref, rstd_ref, wp_ref, bp_ref, dstr_ref,
              x0_ref, flg_ref):
    x = (obs_ref[...] - mean_ref[...]) * rstd_ref[...]
    x0_ref[...] = jnp.tanh(
        jnp.dot(x, wp_ref[...], preferred_element_type=jnp.float32) + bp_ref[...])
    hit = (dstr_ref[...] % _NPG == 0).astype(jnp.int32)
    flg_ref[0, 0, :] = jnp.max(hit, axis=1)


def _pre(obs, rn_mean, rn_rstd, W_pre, b_pre, dst_r):
    blk = 1000
    cblk = _NCHG // (_N // blk)
    return pl.pallas_call(
        _pre_body,
        grid=(_N // blk,),
        in_specs=[
            pl.BlockSpec((blk, _D), lambda i: (i, 0)),
            pl.BlockSpec((_D,), lambda i: (0,)),
            pl.BlockSpec((_D,), lambda i: (0,)),
            pl.BlockSpec((_D, _D), lambda i: (0, 0)),
            pl.BlockSpec((_D,), lambda i: (0,)),
            pl.BlockSpec((cblk, _K), lambda i: (i, 0)),
        ],
        out_specs=[
            pl.BlockSpec((blk, _D), lambda i: (i, 0)),
            pl.BlockSpec((1, 1, cblk), lambda i: (i, 0, 0)),
        ],
        out_shape=[
            jax.ShapeDtypeStruct((_N, _D), jnp.float32),
            jax.ShapeDtypeStruct((_N // blk, 1, cblk), jnp.int32),
        ],
    )(obs, rn_mean, rn_rstd, W_pre, b_pre, dst_r)


_KB = 128
_NF = _EPT // _KB
_TAIL = _EPT - _NF * _KB


def _agg1_body(src_hbm, dst_hbm, x0_hbm, agg_hbm, gsem, ssem, dsem, acc):
    def inner(sidxb, didxw, didxt, rows):
        cid = lax.axis_index("c")
        sid = lax.axis_index("s")
        w = cid * _NS + sid
        base = w * _EPT
        rbase = sid * _RPT

        @pl.loop(0, _KB)
        def _(r):
            for c in range(_D // 16):
                rows[0, r, pl.ds(c * 16, 16)] = jnp.zeros((16,), jnp.float32)

        @pl.loop(0, _RPT // _KB)
        def _(q):
            pltpu.sync_copy(rows.at[0], acc.at[pl.ds(rbase + q * _KB, _KB)])

        pltpu.sync_copy(src_hbm.at[pl.ds(base, _EPT)], sidxb)
        plsc.subcore_barrier()

        def dd(k, j):
            pltpu.async_copy(dst_hbm.at[pl.ds(base + k * _KB, _KB)],
                             didxw.at[j], dsem.at[j])

        def gath(k, j):
            pltpu.async_copy(
                x0_hbm.at[sidxb.at[pl.ds(k * _KB, _KB)]], rows.at[j], gsem.at[j])

        def scat(k, j):
            pltpu.async_copy(rows.at[j], acc.at[didxw.at[j]], ssem.at[j],
                             add=True)

        def wg(j):
            pltpu.make_async_copy(x0_hbm.at[sidxb.at[pl.ds(0, _KB)]],
                                  rows.at[j], gsem.at[j]).wait()

        def ws(j):
            pltpu.make_async_copy(rows.at[j], acc.at[didxw.at[0]],
                                  ssem.at[j]).wait()

        def wd(j):
            pltpu.make_async_copy(dst_hbm.at[pl.ds(base, _KB)], didxw.at[j],
                                  dsem.at[j]).wait()

        dd(0, 0)
        dd(1, 1)
        gath(0, 0)
        gath(1, 1)

        @pl.loop(0, _NF - 2, step=2)
        def _(o):
            wg(0)
            wd(0)
            scat(o, 0)
            wg(1)
            wd(1)
            scat(o + 1, 1)
            ws(0)
            gath(o + 2, 0)
            dd(o + 2, 0)
            ws(1)
            gath(o + 3, 1)
            dd(o + 3, 1)

        wg(0)
        wd(0)
        scat(_NF - 2, 0)
        wg(1)
        wd(1)
        scat(_NF - 1, 1)
        ws(0)
        ws(1)

        pltpu.sync_copy(dst_hbm.at[pl.ds(base + _NF * _KB, _TAIL)], didxt)
        pltpu.sync_copy(x0_hbm.at[sidxb.at[pl.ds(_NF * _KB, _TAIL)]],
                        rows.at[0].at[pl.ds(0, _TAIL)])
        pltpu.sync_copy(rows.at[0].at[pl.ds(0, _TAIL)], acc.at[didxt],
                        add=True)
        plsc.subcore_barrier()

        @pl.loop(0, _RPT // _KB)
        def _(q):
            pltpu.sync_copy(acc.at[pl.ds(rbase + q * _KB, _KB)],
                            agg_hbm.at[cid, pl.ds(rbase + q * _KB, _KB)])

    pl.run_scoped(
        inner,
        pltpu.VMEM((_EPT,), jnp.int32),
        pltpu.VMEM((2, _KB), jnp.int32),
        pltpu.VMEM((_TAIL,), jnp.int32),
        pltpu.VMEM((2, _KB, _D), jnp.float32),
    )


def _agg1(src, dst, x0):
    mesh = plsc.VectorSubcoreMesh(core_axis_name="c", subcore_axis_name="s")
    return pl.kernel(
        _agg1_body,
        out_type=jax.ShapeDtypeStruct((_NC, _NP, _D), jnp.float32),
        mesh=mesh,
        scratch_types=[
            pltpu.SemaphoreType.DMA((2,)),
            pltpu.SemaphoreType.DMA((2,)),
            pltpu.SemaphoreType.DMA((2,)),
            pltpu.VMEM_SHARED((_NP, _D), jnp.float32),
        ],
    )(src, dst, x0)


def _l1_body(x0_ref, agg_ref, ws_ref, wa_ref, b_ref, out_ref):
    a = agg_ref[0] + agg_ref[1]
    out_ref[...] = jnp.tanh(
        jnp.dot(x0_ref[...], ws_ref[...], preferred_element_type=jnp.float32)
        + jnp.dot(a, wa_ref[...], preferred_element_type=jnp.float32)
        + b_ref[...])


def _l1(x0, agg, Ws1, Wa1, b1):
    blk = 1000
    return pl.pallas_call(
        _l1_body,
        grid=(_N // blk,),
        in_specs=[
            pl.BlockSpec((blk, _D), lambda i: (i, 0)),
            pl.BlockSpec((_NC, blk, _D), lambda i: (0, i, 0)),
            pl.BlockSpec((_D, _D), lambda i: (0, 0)),
            pl.BlockSpec((_D, _D), lambda i: (0, 0)),
            pl.BlockSpec((_D,), lambda i: (0,)),
        ],
        out_specs=pl.BlockSpec((blk, _D), lambda i: (i, 0)),
        out_shape=jax.ShapeDtypeStruct((_N, _D), jnp.float32),
    )(x0, agg, Ws1, Wa1, b1)


def _agg2_body(src_hbm, dst_hbm, x1_hbm, flags_hbm, agg2_hbm, acc2):
    def inner(didxc, sidx, gidx, rows, flgv):
        cid = lax.axis_index("c")
        sid = lax.axis_index("s")
        w = cid * _NS + sid
        base = w * _EPT
        for r in range(16):
            for c in range(_D // 16):
                rows[r, pl.ds(c * 16, 16)] = jnp.zeros((16,), jnp.float32)
        pltpu.sync_copy(rows.at[pl.ds(0, 16)], acc2)
        pltpu.sync_copy(flags_hbm.at[pl.ds(w * 128, 128)],
                        flgv.at[pl.ds(0, 128)])
        plsc.subcore_barrier()

        def step(i, carry):
            fv = flgv[pl.ds(i, 16)]

            @pl.when(fv[0] > 0)
            def _():
                pltpu.sync_copy(dst_hbm.at[pl.ds(base + i * _K, _K)], didxc)
                pltpu.sync_copy(src_hbm.at[pl.ds(base + i * _K, _K)], sidx)
                for c in range(_K // 16):
                    d = didxc[pl.ds(c * 16, 16)]
                    s0 = sidx[pl.ds(c * 16, 16)]
                    m = (d % _NPG) == 0
                    sidx[pl.ds(c * 16, 16)] = jnp.where(m, s0, 0)
                    g = (d.astype(jnp.float32) * (1.0 / _NPG)
                         + 0.5).astype(jnp.int32)
                    gidx[pl.ds(c * 16, 16)] = jnp.where(m, g, _B)
                pltpu.sync_copy(x1_hbm.at[sidx], rows)
                pltpu.sync_copy(rows, acc2.at[gidx], add=True)

            return carry

        lax.fori_loop(0, _NCHUNK, step, 0)
        plsc.subcore_barrier()

        @pl.when(sid == 0)
        def _():
            pltpu.sync_copy(acc2, agg2_hbm.at[cid])

    pl.run_scoped(
        inner,
        pltpu.VMEM((_K,), jnp.int32),
        pltpu.VMEM((_K,), jnp.int32),
        pltpu.VMEM((_K,), jnp.int32),
        pltpu.VMEM((_K, _D), jnp.float32),
        pltpu.VMEM((144,), jnp.int32),
    )


def _agg2(src, dst, x1, flags):
    mesh = plsc.VectorSubcoreMesh(core_axis_name="c", subcore_axis_name="s")
    return pl.kernel(
        _agg2_body,
        out_type=jax.ShapeDtypeStruct((_NC, 16, _D), jnp.float32),
        mesh=mesh,
        scratch_types=[
            pltpu.VMEM_SHARED((16, _D), jnp.float32),
        ],
    )(src, dst, x1, flags)


def _tail_body(x1f_ref, agg2_ref, hf_ref, ob_ref, go_ref,
               ws2_ref, wa2_ref, b2_ref, wm_ref, bm_ref,
               wh_ref, bh_ref, wo_ref, bo_ref, wg_ref, bg_ref,
               wv_ref, bv_ref, out_ref):
    dot = functools.partial(jnp.dot, preferred_element_type=jnp.float32)
    agg2 = agg2_ref[0, :_B, :] + agg2_ref[1, :_B, :]
    x2 = jnp.tanh(dot(x1f_ref[...], ws2_ref[...]) + dot(agg2, wa2_ref[...])
                  + b2_ref[...])
    x3 = jnp.tanh(dot(x2, wm_ref[...]) + bm_ref[...])
    xh = jnp.tanh(dot(hf_ref[...], wh_ref[...]) + bh_ref[...])
    xo = jnp.tanh(dot(ob_ref[...], wo_ref[...]) + bo_ref[...])
    xg = jnp.tanh(dot(go_ref[...], wg_ref[...]) + bg_ref[...])
    cat = jnp.concatenate([x3, xh, xo, xg], axis=-1)
    out_ref[...] = dot(cat, wv_ref[...]) + bv_ref[...]


def _tail(x1f, agg2, hfield, obj, goal, Ws2, Wa2, b2, W_mlp, b_mlp,
          Wh, bh, Wo, bo, Wg, bg, Wv, bv):
    return pl.pallas_call(
        _tail_body,
        out_shape=jax.ShapeDtypeStruct((_B, 1), jnp.float32),
    )(x1f, agg2, hfield, obj, goal, Ws2, Wa2, b2, W_mlp, b_mlp,
      Wh, bh, Wo, bo, Wg, bg, Wv, bv)


def kernel(obs, edges, hfield, obj, goal, rn_mean, rn_rstd, W_pre, b_pre,
           Ws1, Wa1, b1, Ws2, Wa2, b2, W_mlp, b_mlp,
           Wh, bh, Wo, bo, Wg, bg, Wv, bv):
    src, dst = edges[0], edges[1]
    dst_r = dst.reshape(_NCHG, _K)
    x0, flags = _pre(obs, rn_mean, rn_rstd, W_pre, b_pre, dst_r)
    flags_p = jnp.pad(flags.reshape(_NW, _NCHUNK), ((0, 0), (0, 3))).reshape(-1)

    agg = _agg1(src, dst, x0)
    x1 = _l1(x0, agg, Ws1, Wa1, b1)
    agg2 = _agg2(src, dst, x1, flags_p)
    x1f = x1[::_NPG]
    return _tail(x1f, agg2, hfield, obj, goal, Ws2, Wa2, b2, W_mlp, b_mlp,
                 Wh, bh, Wo, bo, Wg, bg, Wv, bv)


# speedup vs baseline: 1.0383x; 1.0383x over previous
import functools

import jax
import jax.numpy as jnp
from jax import lax
from jax.experimental import pallas as pl
from jax.experimental.pallas import tpu as pltpu
from jax.experimental.pallas import tpu_sc as plsc

_N = 10000
_E = 320000
_D = 128
_B = 8
_NPG = _N // _B
_NC = 2
_NS = 16
_NW = _NC * _NS
_EPT = _E // _NW
_K = 80
_NCHUNK = _EPT // _K
_NCHG = _E // _K
_NP = 10240
_RPT = _NP // _NS


def _pre_body(obs_ref, mean_ref, rstd_ref, wp_ref, bp_ref, dstr_ref,
              x0_ref, flg_ref):
    x = (obs_ref[...] - mean_ref[...]) * rstd_ref[...]
    x0_ref[...] = jnp.tanh(
        jnp.dot(x, wp_ref[...], preferred_element_type=jnp.float32) + bp_ref[...])
    hit = (dstr_ref[...] % _NPG == 0).astype(jnp.int32)
    flg_ref[0, 0, :] = jnp.max(hit, axis=1)


def _pre(obs, rn_mean, rn_rstd, W_pre, b_pre, dst_r):
    blk = 1000
    cblk = _NCHG // (_N // blk)
    return pl.pallas_call(
        _pre_body,
        grid=(_N // blk,),
        in_specs=[
            pl.BlockSpec((blk, _D), lambda i: (i, 0)),
            pl.BlockSpec((_D,), lambda i: (0,)),
            pl.BlockSpec((_D,), lambda i: (0,)),
            pl.BlockSpec((_D, _D), lambda i: (0, 0)),
            pl.BlockSpec((_D,), lambda i: (0,)),
            pl.BlockSpec((cblk, _K), lambda i: (i, 0)),
        ],
        out_specs=[
            pl.BlockSpec((blk, _D), lambda i: (i, 0)),
            pl.BlockSpec((1, 1, cblk), lambda i: (i, 0, 0)),
        ],
        out_shape=[
            jax.ShapeDtypeStruct((_N, _D), jnp.float32),
            jax.ShapeDtypeStruct((_N // blk, 1, cblk), jnp.int32),
        ],
    )(obs, rn_mean, rn_rstd, W_pre, b_pre, dst_r)


_KB = 128
_NF = _EPT // _KB
_TAIL = _EPT - _NF * _KB


def _agg1_body(src_hbm, dst_hbm, x0_hbm, agg_hbm, gsem, ssem, dsem, acc):
    def inner(sidxb, didxw, didxt, rows):
        cid = lax.axis_index("c")
        sid = lax.axis_index("s")
        w = cid * _NS + sid
        base = w * _EPT
        rbase = sid * _RPT

        @pl.loop(0, _KB)
        def _(r):
            for c in range(_D // 16):
                rows[0, r, pl.ds(c * 16, 16)] = jnp.zeros((16,), jnp.float32)

        @pl.loop(0, _RPT // _KB)
        def _(q):
            pltpu.sync_copy(rows.at[0], acc.at[pl.ds(rbase + q * _KB, _KB)])

        pltpu.sync_copy(src_hbm.at[pl.ds(base, _EPT)], sidxb)
        plsc.subcore_barrier()

        def dd(k, j):
            pltpu.async_copy(dst_hbm.at[pl.ds(base + k * _KB, _KB)],
                             didxw.at[j], dsem.at[j])

        def gath(k, j):
            pltpu.async_copy(
                x0_hbm.at[sidxb.at[pl.ds(k * _KB, _KB)]], rows.at[j], gsem.at[j])

        def scat(k, j):
            pltpu.async_copy(rows.at[j], acc.at[didxw.at[j]], ssem.at[j],
                             add=True)

        def wg(j):
            pltpu.make_async_copy(x0_hbm.at[sidxb.at[pl.ds(0, _KB)]],
                                  rows.at[j], gsem.at[j]).wait()

        def ws(j):
            pltpu.make_async_copy(rows.at[j], acc.at[didxw.at[0]],
                                  ssem.at[j]).wait()

        def wd(j):
            pltpu.make_async_copy(dst_hbm.at[pl.ds(base, _KB)], didxw.at[j],
                                  dsem.at[j]).wait()

        dd(0, 0)
        dd(1, 1)
        gath(0, 0)
        gath(1, 1)

        @pl.loop(0, _NF - 2, step=2)
        def _(o):
            wg(0)
            wd(0)
            scat(o, 0)
            ws(0)
            gath(o + 2, 0)
            dd(o + 2, 0)
            wg(1)
            wd(1)
            scat(o + 1, 1)
            ws(1)
            gath(o + 3, 1)
            dd(o + 3, 1)

        wg(0)
        wd(0)
        scat(_NF - 2, 0)
        ws(0)
        wg(1)
        wd(1)
        scat(_NF - 1, 1)
        ws(1)

        pltpu.sync_copy(dst_hbm.at[pl.ds(base + _NF * _KB, _TAIL)], didxt)
        pltpu.sync_copy(x0_hbm.at[sidxb.at[pl.ds(_NF * _KB, _TAIL)]],
                        rows.at[0].at[pl.ds(0, _TAIL)])
        pltpu.sync_copy(rows.at[0].at[pl.ds(0, _TAIL)], acc.at[didxt],
                        add=True)
        plsc.subcore_barrier()

        @pl.loop(0, _RPT // _KB)
        def _(q):
            pltpu.sync_copy(acc.at[pl.ds(rbase + q * _KB, _KB)],
                            agg_hbm.at[cid, pl.ds(rbase + q * _KB, _KB)])

    pl.run_scoped(
        inner,
        pltpu.VMEM((_EPT,), jnp.int32),
        pltpu.VMEM((2, _KB), jnp.int32),
        pltpu.VMEM((_TAIL,), jnp.int32),
        pltpu.VMEM((2, _KB, _D), jnp.float32),
    )


def _agg1(src, dst, x0):
    mesh = plsc.VectorSubcoreMesh(core_axis_name="c", subcore_axis_name="s")
    return pl.kernel(
        _agg1_body,
        out_type=jax.ShapeDtypeStruct((_NC, _NP, _D), jnp.float32),
        mesh=mesh,
        scratch_types=[
            pltpu.SemaphoreType.DMA((2,)),
            pltpu.SemaphoreType.DMA((2,)),
            pltpu.SemaphoreType.DMA((2,)),
            pltpu.VMEM_SHARED((_NP, _D), jnp.float32),
        ],
    )(src, dst, x0)


def _l1_body(x0_ref, agg_ref, ws_ref, wa_ref, b_ref, out_ref):
    a = agg_ref[0] + agg_ref[1]
    out_ref[...] = jnp.tanh(
        jnp.dot(x0_ref[...], ws_ref[...], preferred_element_type=jnp.float32)
        + jnp.dot(a, wa_ref[...], preferred_element_type=jnp.float32)
        + b_ref[...])


def _l1(x0, agg, Ws1, Wa1, b1):
    blk = 1000
    return pl.pallas_call(
        _l1_body,
        grid=(_N // blk,),
        in_specs=[
            pl.BlockSpec((blk, _D), lambda i: (i, 0)),
            pl.BlockSpec((_NC, blk, _D), lambda i: (0, i, 0)),
            pl.BlockSpec((_D, _D), lambda i: (0, 0)),
            pl.BlockSpec((_D, _D), lambda i: (0, 0)),
            pl.BlockSpec((_D,), lambda i: (0,)),
        ],
        out_specs=pl.BlockSpec((blk, _D), lambda i: (i, 0)),
        out_shape=jax.ShapeDtypeStruct((_N, _D), jnp.float32),
    )(x0, agg, Ws1, Wa1, b1)


def _agg2_body(src_hbm, dst_hbm, x1_hbm, flags_hbm, agg2_hbm,
               dstv, srcv, rows, sidx, gidx, flgv, acc2):
    cid = lax.axis_index("c")
    sid = lax.axis_index("s")
    w = cid * _NS + sid
    base = w * _EPT
    for r in range(16):
        for c in range(_D // 16):
            rows[r, pl.ds(c * 16, 16)] = jnp.zeros((16,), jnp.float32)
    pltpu.sync_copy(rows.at[pl.ds(0, 16)], acc2)
    pltpu.sync_copy(src_hbm.at[pl.ds(base, _EPT)], srcv)
    pltpu.sync_copy(dst_hbm.at[pl.ds(base, _EPT)], dstv)
    pltpu.sync_copy(flags_hbm.at[pl.ds(w * 128, 128)], flgv.at[pl.ds(0, 128)])
    plsc.subcore_barrier()

    def step(i, carry):
        fv = flgv[pl.ds(i, 16)]

        @pl.when(fv[0] > 0)
        def _():
            for c in range(_K // 16):
                d = dstv[pl.ds(i * _K + c * 16, 16)]
                s0 = srcv[pl.ds(i * _K + c * 16, 16)]
                m = (d % _NPG) == 0
                sidx[pl.ds(c * 16, 16)] = jnp.where(m, s0, 0)
                g = (d.astype(jnp.float32) * (1.0 / _NPG) + 0.5).astype(jnp.int32)
                gidx[pl.ds(c * 16, 16)] = jnp.where(m, g, _B)
            pltpu.sync_copy(x1_hbm.at[sidx], rows)
            pltpu.sync_copy(rows, acc2.at[gidx], add=True)

        return carry

    lax.fori_loop(0, _NCHUNK, step, 0)
    plsc.subcore_barrier()

    @pl.when(sid == 0)
    def _():
        pltpu.sync_copy(acc2, agg2_hbm.at[cid])


def _agg2(src, dst, x1, flags):
    mesh = plsc.VectorSubcoreMesh(core_axis_name="c", subcore_axis_name="s")
    return pl.kernel(
        _agg2_body,
        out_type=jax.ShapeDtypeStruct((_NC, 16, _D), jnp.float32),
        mesh=mesh,
        scratch_types=[
            pltpu.VMEM((_EPT,), jnp.int32),
            pltpu.VMEM((_EPT,), jnp.int32),
            pltpu.VMEM((_K, _D), jnp.float32),
            pltpu.VMEM((_K,), jnp.int32),
            pltpu.VMEM((_K,), jnp.int32),
            pltpu.VMEM((144,), jnp.int32),
            pltpu.VMEM_SHARED((16, _D), jnp.float32),
        ],
    )(src, dst, x1, flags)


def _tail_body(x1f_ref, agg2_ref, hf_ref, ob_ref, go_ref,
               ws2_ref, wa2_ref, b2_ref, wm_ref, bm_ref,
               wh_ref, bh_ref, wo_ref, bo_ref, wg_ref, bg_ref,
               wv_ref, bv_ref, out_ref):
    dot = functools.partial(jnp.dot, preferred_element_type=jnp.float32)
    agg2 = agg2_ref[0, :_B, :] + agg2_ref[1, :_B, :]
    x2 = jnp.tanh(dot(x1f_ref[...], ws2_ref[...]) + dot(agg2, wa2_ref[...])
                  + b2_ref[...])
    x3 = jnp.tanh(dot(x2, wm_ref[...]) + bm_ref[...])
    xh = jnp.tanh(dot(hf_ref[...], wh_ref[...]) + bh_ref[...])
    xo = jnp.tanh(dot(ob_ref[...], wo_ref[...]) + bo_ref[...])
    xg = jnp.tanh(dot(go_ref[...], wg_ref[...]) + bg_ref[...])
    cat = jnp.concatenate([x3, xh, xo, xg], axis=-1)
    out_ref[...] = dot(cat, wv_ref[...]) + bv_ref[...]


def _tail(x1f, agg2, hfield, obj, goal, Ws2, Wa2, b2, W_mlp, b_mlp,
          Wh, bh, Wo, bo, Wg, bg, Wv, bv):
    return pl.pallas_call(
        _tail_body,
        out_shape=jax.ShapeDtypeStruct((_B, 1), jnp.float32),
    )(x1f, agg2, hfield, obj, goal, Ws2, Wa2, b2, W_mlp, b_mlp,
      Wh, bh, Wo, bo, Wg, bg, Wv, bv)


def kernel(obs, edges, hfield, obj, goal, rn_mean, rn_rstd, W_pre, b_pre,
           Ws1, Wa1, b1, Ws2, Wa2, b2, W_mlp, b_mlp,
           Wh, bh, Wo, bo, Wg, bg, Wv, bv):
    src, dst = edges[0], edges[1]
    dst_r = dst.reshape(_NCHG, _K)
    x0, flags = _pre(obs, rn_mean, rn_rstd, W_pre, b_pre, dst_r)
    flags_p = jnp.pad(flags.reshape(_NW, _NCHUNK), ((0, 0), (0, 3))).reshape(-1)

    agg = _agg1(src, dst, x0)
    x1 = _l1(x0, agg, Ws1, Wa1, b1)
    agg2 = _agg2(src, dst, x1, flags_p)
    x1f = x1[::_NPG]
    return _tail(x1f, agg2, hfield, obj, goal, Ws2, Wa2, b2, W_mlp, b_mlp,
                 Wh, bh, Wo, bo, Wg, bg, Wv, bv)


# speedup vs baseline: 1.0450x; 1.0064x over previous
import functools

import jax
import jax.numpy as jnp
from jax import lax
from jax.experimental import pallas as pl
from jax.experimental.pallas import tpu as pltpu
from jax.experimental.pallas import tpu_sc as plsc

_N = 10000
_E = 320000
_D = 128
_B = 8
_NPG = _N // _B
_NC = 2
_NS = 16
_NW = _NC * _NS
_EPT = _E // _NW
_K = 80
_NCHUNK = _EPT // _K
_NCHG = _E // _K
_NP = 10240
_RPT = _NP // _NS


def _pre_body(obs_ref, mean_ref, rstd_ref, wp_ref, bp_ref, dstr_ref,
              x0_ref, flg_ref):
    x = (obs_ref[...] - mean_ref[...]) * rstd_ref[...]
    x0_ref[...] = jnp.tanh(
        jnp.dot(x, wp_ref[...], preferred_element_type=jnp.float32) + bp_ref[...])
    hit = (dstr_ref[...] % _NPG == 0).astype(jnp.int32)
    flg_ref[0, 0, :] = jnp.max(hit, axis=1)


def _pre(obs, rn_mean, rn_rstd, W_pre, b_pre, dst_r):
    blk = 2000
    cblk = _NCHG // (_N // blk)
    return pl.pallas_call(
        _pre_body,
        grid=(_N // blk,),
        in_specs=[
            pl.BlockSpec((blk, _D), lambda i: (i, 0)),
            pl.BlockSpec((_D,), lambda i: (0,)),
            pl.BlockSpec((_D,), lambda i: (0,)),
            pl.BlockSpec((_D, _D), lambda i: (0, 0)),
            pl.BlockSpec((_D,), lambda i: (0,)),
            pl.BlockSpec((cblk, _K), lambda i: (i, 0)),
        ],
        out_specs=[
            pl.BlockSpec((blk, _D), lambda i: (i, 0)),
            pl.BlockSpec((1, 1, cblk), lambda i: (i, 0, 0)),
        ],
        out_shape=[
            jax.ShapeDtypeStruct((_N, _D), jnp.float32),
            jax.ShapeDtypeStruct((_N // blk, 1, cblk), jnp.int32),
        ],
    )(obs, rn_mean, rn_rstd, W_pre, b_pre, dst_r)


_KB = 128
_NF = _EPT // _KB
_TAIL = _EPT - _NF * _KB


def _agg1_body(src_hbm, dst_hbm, x0_hbm, agg_hbm, gsem, ssem, dsem, acc):
    def inner(sidxb, didxw, didxt, rows):
        cid = lax.axis_index("c")
        sid = lax.axis_index("s")
        w = cid * _NS + sid
        base = w * _EPT
        rbase = sid * _RPT

        @pl.loop(0, _KB)
        def _(r):
            for c in range(_D // 16):
                rows[0, r, pl.ds(c * 16, 16)] = jnp.zeros((16,), jnp.float32)

        @pl.loop(0, _RPT // _KB)
        def _(q):
            pltpu.async_copy(rows.at[0], acc.at[pl.ds(rbase + q * _KB, _KB)],
                             gsem.at[0])

        pltpu.sync_copy(src_hbm.at[pl.ds(base, _EPT)], sidxb)

        @pl.loop(0, _RPT // _KB)
        def _(q):
            pltpu.make_async_copy(rows.at[0],
                                  acc.at[pl.ds(rbase + q * _KB, _KB)],
                                  gsem.at[0]).wait()
        plsc.subcore_barrier()

        def dd(k, j):
            pltpu.async_copy(dst_hbm.at[pl.ds(base + k * _KB, _KB)],
                             didxw.at[j], dsem.at[j])

        def gath(k, j):
            pltpu.async_copy(
                x0_hbm.at[sidxb.at[pl.ds(k * _KB, _KB)]], rows.at[j], gsem.at[j])

        def scat(k, j):
            pltpu.async_copy(rows.at[j], acc.at[didxw.at[j]], ssem.at[j],
                             add=True)

        def wg(j):
            pltpu.make_async_copy(x0_hbm.at[sidxb.at[pl.ds(0, _KB)]],
                                  rows.at[j], gsem.at[j]).wait()

        def ws(j):
            pltpu.make_async_copy(rows.at[j], acc.at[didxw.at[0]],
                                  ssem.at[j]).wait()

        def wd(j):
            pltpu.make_async_copy(dst_hbm.at[pl.ds(base, _KB)], didxw.at[j],
                                  dsem.at[j]).wait()

        dd(0, 0)
        dd(1, 1)
        gath(0, 0)
        gath(1, 1)

        @pl.loop(0, _NF - 2, step=2)
        def _(o):
            wg(0)
            wd(0)
            scat(o, 0)
            ws(0)
            gath(o + 2, 0)
            dd(o + 2, 0)
            wg(1)
            wd(1)
            scat(o + 1, 1)
            ws(1)
            gath(o + 3, 1)
            dd(o + 3, 1)

        wg(0)
        wd(0)
        scat(_NF - 2, 0)
        ws(0)
        wg(1)
        wd(1)
        scat(_NF - 1, 1)
        ws(1)

        pltpu.sync_copy(dst_hbm.at[pl.ds(base + _NF * _KB, _TAIL)], didxt)
        pltpu.sync_copy(x0_hbm.at[sidxb.at[pl.ds(_NF * _KB, _TAIL)]],
                        rows.at[0].at[pl.ds(0, _TAIL)])
        pltpu.sync_copy(rows.at[0].at[pl.ds(0, _TAIL)], acc.at[didxt],
                        add=True)
        plsc.subcore_barrier()

        @pl.loop(0, _RPT // _KB)
        def _(q):
            pltpu.sync_copy(acc.at[pl.ds(rbase + q * _KB, _KB)],
                            agg_hbm.at[cid, pl.ds(rbase + q * _KB, _KB)])

    pl.run_scoped(
        inner,
        pltpu.VMEM((_EPT,), jnp.int32),
        pltpu.VMEM((2, _KB), jnp.int32),
        pltpu.VMEM((_TAIL,), jnp.int32),
        pltpu.VMEM((2, _KB, _D), jnp.float32),
    )


def _agg1(src, dst, x0):
    mesh = plsc.VectorSubcoreMesh(core_axis_name="c", subcore_axis_name="s")
    return pl.kernel(
        _agg1_body,
        out_type=jax.ShapeDtypeStruct((_NC, _NP, _D), jnp.float32),
        mesh=mesh,
        scratch_types=[
            pltpu.SemaphoreType.DMA((2,)),
            pltpu.SemaphoreType.DMA((2,)),
            pltpu.SemaphoreType.DMA((2,)),
            pltpu.VMEM_SHARED((_NP, _D), jnp.float32),
        ],
    )(src, dst, x0)


def _l1_body(x0_ref, agg_ref, ws_ref, wa_ref, b_ref, out_ref):
    a = agg_ref[0] + agg_ref[1]
    out_ref[...] = jnp.tanh(
        jnp.dot(x0_ref[...], ws_ref[...], preferred_element_type=jnp.float32)
        + jnp.dot(a, wa_ref[...], preferred_element_type=jnp.float32)
        + b_ref[...])


def _l1(x0, agg, Ws1, Wa1, b1):
    blk = 2000
    return pl.pallas_call(
        _l1_body,
        grid=(_N // blk,),
        in_specs=[
            pl.BlockSpec((blk, _D), lambda i: (i, 0)),
            pl.BlockSpec((_NC, blk, _D), lambda i: (0, i, 0)),
            pl.BlockSpec((_D, _D), lambda i: (0, 0)),
            pl.BlockSpec((_D, _D), lambda i: (0, 0)),
            pl.BlockSpec((_D,), lambda i: (0,)),
        ],
        out_specs=pl.BlockSpec((blk, _D), lambda i: (i, 0)),
        out_shape=jax.ShapeDtypeStruct((_N, _D), jnp.float32),
    )(x0, agg, Ws1, Wa1, b1)


def _agg2_body(src_hbm, dst_hbm, x1_hbm, flags_hbm, agg2_hbm,
               dstv, srcv, rows, sidx, gidx, flgv, acc2):
    cid = lax.axis_index("c")
    sid = lax.axis_index("s")
    w = cid * _NS + sid
    base = w * _EPT
    for r in range(16):
        for c in range(_D // 16):
            rows[r, pl.ds(c * 16, 16)] = jnp.zeros((16,), jnp.float32)
    pltpu.sync_copy(rows.at[pl.ds(0, 16)], acc2)
    pltpu.sync_copy(src_hbm.at[pl.ds(base, _EPT)], srcv)
    pltpu.sync_copy(dst_hbm.at[pl.ds(base, _EPT)], dstv)
    pltpu.sync_copy(flags_hbm.at[pl.ds(w * 128, 128)], flgv.at[pl.ds(0, 128)])
    plsc.subcore_barrier()

    def step(i, carry):
        fv = flgv[pl.ds(i, 16)]

        @pl.when(fv[0] > 0)
        def _():
            for c in range(_K // 16):
                d = dstv[pl.ds(i * _K + c * 16, 16)]
                s0 = srcv[pl.ds(i * _K + c * 16, 16)]
                m = (d % _NPG) == 0
                sidx[pl.ds(c * 16, 16)] = jnp.where(m, s0, 0)
                g = (d.astype(jnp.float32) * (1.0 / _NPG) + 0.5).astype(jnp.int32)
                gidx[pl.ds(c * 16, 16)] = jnp.where(m, g, _B)
            pltpu.sync_copy(x1_hbm.at[sidx], rows)
            pltpu.sync_copy(rows, acc2.at[gidx], add=True)

        return carry

    lax.fori_loop(0, _NCHUNK, step, 0)
    plsc.subcore_barrier()

    @pl.when(sid == 0)
    def _():
        pltpu.sync_copy(acc2, agg2_hbm.at[cid])


def _agg2(src, dst, x1, flags):
    mesh = plsc.VectorSubcoreMesh(core_axis_name="c", subcore_axis_name="s")
    return pl.kernel(
        _agg2_body,
        out_type=jax.ShapeDtypeStruct((_NC, 16, _D), jnp.float32),
        mesh=mesh,
        scratch_types=[
            pltpu.VMEM((_EPT,), jnp.int32),
            pltpu.VMEM((_EPT,), jnp.int32),
            pltpu.VMEM((_K, _D), jnp.float32),
            pltpu.VMEM((_K,), jnp.int32),
            pltpu.VMEM((_K,), jnp.int32),
            pltpu.VMEM((144,), jnp.int32),
            pltpu.VMEM_SHARED((16, _D), jnp.float32),
        ],
    )(src, dst, x1, flags)


def _tail_body(x1f_ref, agg2_ref, hf_ref, ob_ref, go_ref,
               ws2_ref, wa2_ref, b2_ref, wm_ref, bm_ref,
               wh_ref, bh_ref, wo_ref, bo_ref, wg_ref, bg_ref,
               wv_ref, bv_ref, out_ref):
    dot = functools.partial(jnp.dot, preferred_element_type=jnp.float32)
    agg2 = agg2_ref[0, :_B, :] + agg2_ref[1, :_B, :]
    x2 = jnp.tanh(dot(x1f_ref[...], ws2_ref[...]) + dot(agg2, wa2_ref[...])
                  + b2_ref[...])
    x3 = jnp.tanh(dot(x2, wm_ref[...]) + bm_ref[...])
    xh = jnp.tanh(dot(hf_ref[...], wh_ref[...]) + bh_ref[...])
    xo = jnp.tanh(dot(ob_ref[...], wo_ref[...]) + bo_ref[...])
    xg = jnp.tanh(dot(go_ref[...], wg_ref[...]) + bg_ref[...])
    cat = jnp.concatenate([x3, xh, xo, xg], axis=-1)
    out_ref[...] = dot(cat, wv_ref[...]) + bv_ref[...]


def _tail(x1f, agg2, hfield, obj, goal, Ws2, Wa2, b2, W_mlp, b_mlp,
          Wh, bh, Wo, bo, Wg, bg, Wv, bv):
    return pl.pallas_call(
        _tail_body,
        out_shape=jax.ShapeDtypeStruct((_B, 1), jnp.float32),
    )(x1f, agg2, hfield, obj, goal, Ws2, Wa2, b2, W_mlp, b_mlp,
      Wh, bh, Wo, bo, Wg, bg, Wv, bv)


def kernel(obs, edges, hfield, obj, goal, rn_mean, rn_rstd, W_pre, b_pre,
           Ws1, Wa1, b1, Ws2, Wa2, b2, W_mlp, b_mlp,
           Wh, bh, Wo, bo, Wg, bg, Wv, bv):
    src, dst = edges[0], edges[1]
    dst_r = dst.reshape(_NCHG, _K)
    x0, flags = _pre(obs, rn_mean, rn_rstd, W_pre, b_pre, dst_r)
    flags_p = jnp.pad(flags.reshape(_NW, _NCHUNK), ((0, 0), (0, 3))).reshape(-1)

    agg = _agg1(src, dst, x0)
    x1 = _l1(x0, agg, Ws1, Wa1, b1)
    agg2 = _agg2(src, dst, x1, flags_p)
    x1f = x1[::_NPG]
    return _tail(x1f, agg2, hfield, obj, goal, Ws2, Wa2, b2, W_mlp, b_mlp,
                 Wh, bh, Wo, bo, Wg, bg, Wv, bv)
